# baseline (speedup 1.0000x reference)
import functools

import jax
import jax.numpy as jnp
from jax import lax
from jax.experimental import pallas as pl
from jax.experimental.pallas import tpu as pltpu
from jax.experimental.pallas import tpu_sc as plsc

BATCH = 16384
DIM = 32
_NC = 2
_NS = 16
_NW = _NC * _NS
_BPW = BATCH // _NW

_mesh = plsc.VectorSubcoreMesh(core_axis_name="c", subcore_axis_name="s")


@functools.partial(
    pl.kernel,
    mesh=_mesh,
    out_type=[
        jax.ShapeDtypeStruct((BATCH, DIM), jnp.float32),
        jax.ShapeDtypeStruct((BATCH, DIM), jnp.float32),
        jax.ShapeDtypeStruct((BATCH, DIM), jnp.float32),
    ],
    scratch_types=[
        pltpu.VMEM((_BPW,), jnp.int32),
        pltpu.VMEM((_BPW,), jnp.int32),
        pltpu.VMEM((_BPW,), jnp.int32),
        pltpu.VMEM((_BPW, DIM), jnp.float32),
        pltpu.VMEM((_BPW, DIM), jnp.float32),
        pltpu.VMEM((_BPW, DIM), jnp.float32),
        pltpu.SemaphoreType.DMA,
    ],
    compiler_params=pltpu.CompilerParams(use_tc_tiling_on_sc=False),
)
def _gather3(head_hbm, rel_hbm, tail_hbm, ent_hbm, reltab_hbm,
             out_h, out_r, out_t,
             idx_h, idx_r, idx_t, rows_h, rows_r, rows_t, sem):
    wid = lax.axis_index("s") * _NC + lax.axis_index("c")
    base = wid * _BPW
    sl = pl.ds(base, _BPW)
    pltpu.sync_copy(head_hbm.at[sl], idx_h)
    pltpu.sync_copy(rel_hbm.at[sl], idx_r)
    pltpu.sync_copy(tail_hbm.at[sl], idx_t)
    ch = pltpu.async_copy(ent_hbm.at[idx_h], rows_h, sem)
    cr = pltpu.async_copy(reltab_hbm.at[idx_r], rows_r, sem)
    ct = pltpu.async_copy(ent_hbm.at[idx_t], rows_t, sem)
    ch.wait()
    cr.wait()
    ct.wait()
    pltpu.sync_copy(rows_h, out_h.at[sl])
    pltpu.sync_copy(rows_r, out_r.at[sl])
    pltpu.sync_copy(rows_t, out_t.at[sl])


def kernel(raw_triples, entity_embeddings, relation_embeddings):
    tri = raw_triples.astype(jnp.int32)
    head = tri[:, 0]
    rel = tri[:, 1]
    tail = tri[:, 2]
    head_emb, rel_emb, tail_emb = _gather3(
        head, rel, tail, entity_embeddings, relation_embeddings)
    return (head_emb, rel_emb, tail_emb)
